# baseline (speedup 1.0000x reference)
import functools

import jax
import jax.numpy as jnp
from jax.experimental import pallas as pl
from jax.experimental.pallas import tpu as pltpu
from jax.experimental.pallas import tpu_sc as plsc

def _pack32(a):
    n, d = a.shape
    return jax.lax.bitcast_convert_type(
        a.reshape(n, d // 2, 2), jnp.int32)


def _unpack16(a):
    n, d = a.shape
    return jax.lax.bitcast_convert_type(a, jnp.bfloat16).reshape(n, 2 * d)


_RT = 512
_TK = 256
_SCW = 32
_NWORKERS = 32


def _router_body(x_ref, wg_ref, bg_ref, g_ref, e_ref, r_ref, cnt_ref, xb_ref,
                 carry):
    i = pl.program_id(0)

    @pl.when(i == 0)
    def _():
        carry[...] = jnp.zeros_like(carry)

    x = x_ref[...]
    xb_ref[...] = x.astype(jnp.bfloat16)
    logits = jax.lax.dot_general(
        x, wg_ref[...], (((1,), (1,)), ((), ())),
        preferred_element_type=jnp.float32) + bg_ref[...]
    m = jnp.max(logits, axis=1, keepdims=True)
    ex = jnp.exp(logits - m)
    probs = ex / jnp.sum(ex, axis=1, keepdims=True)

    lane = jax.lax.broadcasted_iota(jnp.int32, probs.shape, 1)
    i1 = jnp.argmax(probs, axis=1)
    m1 = lane == i1[:, None]
    p1 = jnp.max(probs, axis=1)
    masked = jnp.where(m1, -jnp.inf, probs)
    i2 = jnp.argmax(masked, axis=1)
    m2 = lane == i2[:, None]
    p2 = jnp.max(masked, axis=1)

    a = m1.astype(jnp.float32) + m2.astype(jnp.float32)
    rows = jax.lax.broadcasted_iota(jnp.int32, (a.shape[0], a.shape[0]), 0)
    cols = jax.lax.broadcasted_iota(jnp.int32, (a.shape[0], a.shape[0]), 1)
    tri = (rows > cols).astype(jnp.float32)
    s = jax.lax.dot_general(tri, a, (((1,), (0,)), ((), ())),
                            preferred_element_type=jnp.float32) + carry[...]
    r0 = jnp.sum(jnp.where(m1, s, 0.0), axis=1)
    r1 = jnp.sum(jnp.where(m2, s, 0.0), axis=1)
    new_carry = carry[...] + jnp.sum(a, axis=0, keepdims=True)
    carry[...] = new_carry
    cnt_ref[...] = new_carry.astype(jnp.int32)

    g_ref[...] = jnp.concatenate([p1[:, None], p2[:, None]], axis=1)
    e_ref[...] = jnp.concatenate(
        [i1[:, None], i2[:, None]], axis=1).astype(jnp.int32)
    r_ref[...] = jnp.concatenate(
        [r0[:, None], r1[:, None]], axis=1).astype(jnp.int32)


def _run_router(xf, Wg, bg):
    n, d = xf.shape
    e = Wg.shape[0]
    nt = n // _RT
    return pl.pallas_call(
        _router_body,
        grid=(nt,),
        in_specs=[
            pl.BlockSpec((_RT, d), lambda i: (i, 0)),
            pl.BlockSpec((e, d), lambda i: (0, 0)),
            pl.BlockSpec((1, e), lambda i: (0, 0)),
        ],
        out_specs=[
            pl.BlockSpec((_RT, 2), lambda i: (i, 0)),
            pl.BlockSpec((_RT, 2), lambda i: (i, 0)),
            pl.BlockSpec((_RT, 2), lambda i: (i, 0)),
            pl.BlockSpec((1, e), lambda i: (0, 0)),
            pl.BlockSpec((_RT, d), lambda i: (i, 0)),
        ],
        out_shape=[
            jax.ShapeDtypeStruct((n, 2), jnp.float32),
            jax.ShapeDtypeStruct((n, 2), jnp.int32),
            jax.ShapeDtypeStruct((n, 2), jnp.int32),
            jax.ShapeDtypeStruct((1, e), jnp.int32),
            jax.ShapeDtypeStruct((n, d), jnp.bfloat16),
        ],
        scratch_shapes=[pltpu.VMEM((1, e), jnp.float32)],
        compiler_params=pltpu.CompilerParams(
            dimension_semantics=("arbitrary",)),
    )(xf, Wg, bg.reshape(1, e))


def _sc_dispatch(xf, s0, s1, p_slots):
    n, d = xf.shape
    b_per_w = n // _NWORKERS
    nch = b_per_w // _SCW
    mesh = plsc.VectorSubcoreMesh(core_axis_name="c", subcore_axis_name="s")

    @functools.partial(
        pl.kernel, mesh=mesh,
        out_type=jax.ShapeDtypeStruct((p_slots, d), xf.dtype),
        scratch_types=[
            pltpu.VMEM((_SCW,), jnp.int32),
            pltpu.VMEM((_SCW,), jnp.int32),
            pltpu.VMEM((_SCW,), jnp.int32),
            pltpu.VMEM((_SCW,), jnp.int32),
            pltpu.VMEM((_SCW, d), jnp.int32),
            pltpu.VMEM((_SCW, d), jnp.int32),
            pltpu.SemaphoreType.DMA,
            pltpu.SemaphoreType.DMA,
            pltpu.SemaphoreType.DMA,
            pltpu.SemaphoreType.DMA,
            pltpu.SemaphoreType.DMA,
            pltpu.SemaphoreType.DMA,
        ])
    def k(x_hbm, s0_hbm, s1_hbm, o_hbm, i0a, i1a, i0b, i1b, ra, rb,
          sla, slb, sa0, sa1, sb0, sb1):
        wid = jax.lax.axis_index("s") * 2 + jax.lax.axis_index("c")
        base = wid * b_per_w

        @pl.loop(0, nch, step=2)
        def _(j):
            offa = base + j * _SCW
            offb = offa + _SCW
            pltpu.sync_copy(s0_hbm.at[pl.ds(offa, _SCW)], i0a)
            pltpu.sync_copy(s1_hbm.at[pl.ds(offa, _SCW)], i1a)
            la = pltpu.async_copy(x_hbm.at[pl.ds(offa, _SCW)], ra, sla)
            pltpu.sync_copy(s0_hbm.at[pl.ds(offb, _SCW)], i0b)
            pltpu.sync_copy(s1_hbm.at[pl.ds(offb, _SCW)], i1b)
            lb = pltpu.async_copy(x_hbm.at[pl.ds(offb, _SCW)], rb, slb)
            la.wait()
            ca0 = pltpu.async_copy(ra, o_hbm.at[i0a], sa0)
            ca1 = pltpu.async_copy(ra, o_hbm.at[i1a], sa1)
            lb.wait()
            cb0 = pltpu.async_copy(rb, o_hbm.at[i0b], sb0)
            cb1 = pltpu.async_copy(rb, o_hbm.at[i1b], sb1)
            ca0.wait()
            ca1.wait()
            cb0.wait()
            cb1.wait()

    return k(xf, s0, s1)


def _sc_combine_gather(y, s0, s1):
    _, d = y.shape
    n = s0.shape[0]
    b_per_w = n // _NWORKERS
    nch = b_per_w // _SCW
    mesh = plsc.VectorSubcoreMesh(core_axis_name="c", subcore_axis_name="s")
    otype = jax.ShapeDtypeStruct((n, d), y.dtype)

    @functools.partial(
        pl.kernel, mesh=mesh,
        out_type=(otype, otype),
        scratch_types=[
            pltpu.VMEM((_SCW,), jnp.int32),
            pltpu.VMEM((_SCW,), jnp.int32),
            pltpu.VMEM((_SCW, d), jnp.int32),
            pltpu.VMEM((_SCW, d), jnp.int32),
            pltpu.SemaphoreType.DMA,
            pltpu.SemaphoreType.DMA,
            pltpu.SemaphoreType.DMA,
            pltpu.SemaphoreType.DMA,
        ])
    def k(y_hbm, s0_hbm, s1_hbm, o0_hbm, o1_hbm, ia, ib, ra, rb,
          sga, sgb, ssa, ssb):
        wid = jax.lax.axis_index("s") * 2 + jax.lax.axis_index("c")
        base = wid * b_per_w

        @pl.loop(0, nch)
        def _(j):
            off = base + j * _SCW
            pltpu.sync_copy(s0_hbm.at[pl.ds(off, _SCW)], ia)
            ga = pltpu.async_copy(y_hbm.at[ia], ra, sga)
            pltpu.sync_copy(s1_hbm.at[pl.ds(off, _SCW)], ib)
            gb = pltpu.async_copy(y_hbm.at[ib], rb, sgb)
            ga.wait()
            sa = pltpu.async_copy(ra, o0_hbm.at[pl.ds(off, _SCW)], ssa)
            gb.wait()
            sb = pltpu.async_copy(rb, o1_hbm.at[pl.ds(off, _SCW)], ssb)
            sa.wait()
            sb.wait()

    return k(y, s0, s1)


def _ffn_body(te_ref, nu_ref, x_ref, w1_ref, b1_ref, w2_ref, b2_ref, y_ref,
              w1b_ref, w2b_ref, last_ref):
    i = pl.program_id(0)

    @pl.when(i < nu_ref[0])
    def _():
        @pl.when((i == 0) | (te_ref[i] != last_ref[0]))
        def _():
            w1b_ref[...] = w1_ref[0].astype(jnp.bfloat16)
            w2b_ref[...] = w2_ref[0].astype(jnp.bfloat16)

        last_ref[0] = te_ref[i]
        h = jax.lax.dot_general(x_ref[...], w1b_ref[...],
                                (((1,), (0,)), ((), ())),
                                preferred_element_type=jnp.float32)
        h = h + b1_ref[0]
        h = 0.5 * h * (1.0 + jax.lax.erf(h * 0.7071067811865476))
        y = jax.lax.dot_general(h.astype(jnp.bfloat16), w2b_ref[...],
                                (((1,), (0,)), ((), ())),
                                preferred_element_type=jnp.float32)
        y_ref[...] = (y + b2_ref[0]).astype(jnp.bfloat16)


def _run_ffn(x_sorted, tile_expert, n_used, W1, b1, W2, b2, nt_pad):
    e, d, hd = W1.shape
    grid_spec = pltpu.PrefetchScalarGridSpec(
        num_scalar_prefetch=2,
        grid=(nt_pad,),
        in_specs=[
            pl.BlockSpec((_TK, d), lambda i, te, nu: (i, 0)),
            pl.BlockSpec((1, d, hd), lambda i, te, nu: (te[i], 0, 0)),
            pl.BlockSpec((1, 1, hd), lambda i, te, nu: (te[i], 0, 0)),
            pl.BlockSpec((1, hd, d), lambda i, te, nu: (te[i], 0, 0)),
            pl.BlockSpec((1, 1, d), lambda i, te, nu: (te[i], 0, 0)),
        ],
        out_specs=pl.BlockSpec((_TK, d), lambda i, te, nu: (i, 0)),
        scratch_shapes=[
            pltpu.VMEM((d, hd), jnp.bfloat16),
            pltpu.VMEM((hd, d), jnp.bfloat16),
            pltpu.SMEM((1,), jnp.int32),
        ],
    )
    return pl.pallas_call(
        _ffn_body,
        grid_spec=grid_spec,
        out_shape=jax.ShapeDtypeStruct((nt_pad * _TK, d), jnp.bfloat16),
        compiler_params=pltpu.CompilerParams(
            dimension_semantics=("arbitrary",)),
    )(tile_expert, n_used, x_sorted, W1, b1.reshape(e, 1, hd), W2,
      b2.reshape(e, 1, d))


def _combine_body(g_ref, y0_ref, y1_ref, o_ref):
    g = g_ref[...]
    o_ref[...] = (g[:, 0:1] * y0_ref[...].astype(jnp.float32)
                  + g[:, 1:2] * y1_ref[...].astype(jnp.float32))


def _run_combine(g, y0, y1):
    n, d = y0.shape
    nt = n // _RT
    return pl.pallas_call(
        _combine_body,
        grid=(nt,),
        in_specs=[
            pl.BlockSpec((_RT, 2), lambda i: (i, 0)),
            pl.BlockSpec((_RT, d), lambda i: (i, 0)),
            pl.BlockSpec((_RT, d), lambda i: (i, 0)),
        ],
        out_specs=pl.BlockSpec((_RT, d), lambda i: (i, 0)),
        out_shape=jax.ShapeDtypeStruct((n, d), jnp.float32),
    )(g, y0, y1)


@jax.jit
def kernel(x, Wg, bg, W1, b1, W2, b2):
    Bx, Tx, Dx = x.shape
    E, D, H = W1.shape
    xf = x.reshape(-1, Dx)
    N = xf.shape[0]
    nt_pad = (2 * N) // _TK + E
    p_slots = nt_pad * _TK

    g, e_idx, rank, counts, xb = _run_router(xf, Wg, bg)

    cnt = counts[0]
    padded = ((cnt + _TK - 1) // _TK) * _TK
    ends = jnp.cumsum(padded)
    offs = ends - padded
    slots = jnp.take(offs, e_idx, axis=0) + rank
    s0 = slots[:, 0].astype(jnp.int32)
    s1 = slots[:, 1].astype(jnp.int32)
    n_used = (ends[E - 1] // _TK).astype(jnp.int32).reshape(1)
    tiles = jnp.arange(nt_pad, dtype=jnp.int32)
    te_raw = jnp.sum((ends[None, :] <= (tiles * _TK)[:, None]).astype(
        jnp.int32), axis=1)
    te_last = te_raw[jnp.clip(n_used[0] - 1, 0, nt_pad - 1)]
    tile_expert = jnp.clip(
        jnp.where(tiles >= n_used[0], te_last, te_raw),
        0, E - 1).astype(jnp.int32)

    x_sorted = _unpack16(_sc_dispatch(_pack32(xb), s0, s1, p_slots))
    y = _run_ffn(x_sorted, tile_expert, n_used, W1, b1, W2, b2, nt_pad)
    y0, y1 = _sc_combine_gather(_pack32(y), s0, s1)
    out = _run_combine(g, _unpack16(y0), _unpack16(y1))
    return out.reshape(Bx, Tx, Dx)


# speedup vs baseline: 3.7836x; 3.7836x over previous
import functools

import jax
import jax.numpy as jnp
from jax.experimental import pallas as pl
from jax.experimental.pallas import tpu as pltpu
from jax.experimental.pallas import tpu_sc as plsc

_RT = 512
_TK = 512
_SCW = 32
_NWORKERS = 32


def _router_body(x_ref, wg_ref, bg_ref, g_ref, e_ref, r_ref, cnt_ref, carry):
    i = pl.program_id(0)

    @pl.when(i == 0)
    def _():
        carry[...] = jnp.zeros_like(carry)

    x = x_ref[...]
    logits = jax.lax.dot_general(
        x, wg_ref[...], (((1,), (1,)), ((), ())),
        preferred_element_type=jnp.float32) + bg_ref[...]
    m = jnp.max(logits, axis=1, keepdims=True)
    ex = jnp.exp(logits - m)
    probs = ex / jnp.sum(ex, axis=1, keepdims=True)

    lane = jax.lax.broadcasted_iota(jnp.int32, probs.shape, 1)
    i1 = jnp.argmax(probs, axis=1)
    m1 = lane == i1[:, None]
    p1 = jnp.max(probs, axis=1)
    masked = jnp.where(m1, -jnp.inf, probs)
    i2 = jnp.argmax(masked, axis=1)
    m2 = lane == i2[:, None]
    p2 = jnp.max(masked, axis=1)

    a = m1.astype(jnp.float32) + m2.astype(jnp.float32)
    rows = jax.lax.broadcasted_iota(jnp.int32, (a.shape[0], a.shape[0]), 0)
    cols = jax.lax.broadcasted_iota(jnp.int32, (a.shape[0], a.shape[0]), 1)
    tri = (rows > cols).astype(jnp.float32)
    s = jax.lax.dot_general(tri, a, (((1,), (0,)), ((), ())),
                            preferred_element_type=jnp.float32) + carry[...]
    r0 = jnp.sum(jnp.where(m1, s, 0.0), axis=1)
    r1 = jnp.sum(jnp.where(m2, s, 0.0), axis=1)
    new_carry = carry[...] + jnp.sum(a, axis=0, keepdims=True)
    carry[...] = new_carry
    cnt_ref[...] = new_carry.astype(jnp.int32)

    g_ref[...] = jnp.concatenate([p1[:, None], p2[:, None]], axis=1)
    e_ref[...] = jnp.concatenate(
        [i1[:, None], i2[:, None]], axis=1).astype(jnp.int32)
    r_ref[...] = jnp.concatenate(
        [r0[:, None], r1[:, None]], axis=1).astype(jnp.int32)


def _run_router(xf, Wg, bg):
    n, d = xf.shape
    e = Wg.shape[0]
    nt = n // _RT
    return pl.pallas_call(
        _router_body,
        grid=(nt,),
        in_specs=[
            pl.BlockSpec((_RT, d), lambda i: (i, 0)),
            pl.BlockSpec((e, d), lambda i: (0, 0)),
            pl.BlockSpec((1, e), lambda i: (0, 0)),
        ],
        out_specs=[
            pl.BlockSpec((_RT, 2), lambda i: (i, 0)),
            pl.BlockSpec((_RT, 2), lambda i: (i, 0)),
            pl.BlockSpec((_RT, 2), lambda i: (i, 0)),
            pl.BlockSpec((1, e), lambda i: (0, 0)),
        ],
        out_shape=[
            jax.ShapeDtypeStruct((n, 2), jnp.float32),
            jax.ShapeDtypeStruct((n, 2), jnp.int32),
            jax.ShapeDtypeStruct((n, 2), jnp.int32),
            jax.ShapeDtypeStruct((1, e), jnp.int32),
        ],
        scratch_shapes=[pltpu.VMEM((1, e), jnp.float32)],
        compiler_params=pltpu.CompilerParams(
            dimension_semantics=("arbitrary",)),
    )(xf, Wg, bg.reshape(1, e))


def _sc_dispatch(xf, s0, s1, p_slots):
    n, d = xf.shape
    b_per_w = n // _NWORKERS
    nch = b_per_w // _SCW
    mesh = plsc.VectorSubcoreMesh(core_axis_name="c", subcore_axis_name="s")

    @functools.partial(
        pl.kernel, mesh=mesh,
        out_type=jax.ShapeDtypeStruct((p_slots, d), xf.dtype),
        scratch_types=[
            pltpu.VMEM((_SCW,), jnp.int32),
            pltpu.VMEM((_SCW,), jnp.int32),
            pltpu.VMEM((_SCW,), jnp.int32),
            pltpu.VMEM((_SCW,), jnp.int32),
            pltpu.VMEM((_SCW, d), jnp.float32),
            pltpu.VMEM((_SCW, d), jnp.float32),
            pltpu.SemaphoreType.DMA,
            pltpu.SemaphoreType.DMA,
            pltpu.SemaphoreType.DMA,
            pltpu.SemaphoreType.DMA,
            pltpu.SemaphoreType.DMA,
            pltpu.SemaphoreType.DMA,
        ])
    def k(x_hbm, s0_hbm, s1_hbm, o_hbm, i0a, i1a, i0b, i1b, ra, rb,
          sla, slb, sa0, sa1, sb0, sb1):
        wid = jax.lax.axis_index("s") * 2 + jax.lax.axis_index("c")
        base = wid * b_per_w

        @pl.loop(0, nch, step=2)
        def _(j):
            offa = base + j * _SCW
            offb = offa + _SCW
            pltpu.sync_copy(s0_hbm.at[pl.ds(offa, _SCW)], i0a)
            pltpu.sync_copy(s1_hbm.at[pl.ds(offa, _SCW)], i1a)
            la = pltpu.async_copy(x_hbm.at[pl.ds(offa, _SCW)], ra, sla)
            pltpu.sync_copy(s0_hbm.at[pl.ds(offb, _SCW)], i0b)
            pltpu.sync_copy(s1_hbm.at[pl.ds(offb, _SCW)], i1b)
            lb = pltpu.async_copy(x_hbm.at[pl.ds(offb, _SCW)], rb, slb)
            la.wait()
            ca0 = pltpu.async_copy(ra, o_hbm.at[i0a], sa0)
            ca1 = pltpu.async_copy(ra, o_hbm.at[i1a], sa1)
            lb.wait()
            cb0 = pltpu.async_copy(rb, o_hbm.at[i0b], sb0)
            cb1 = pltpu.async_copy(rb, o_hbm.at[i1b], sb1)
            ca0.wait()
            ca1.wait()
            cb0.wait()
            cb1.wait()

    return k(xf, s0, s1)


def _sc_combine_gather(y, s0, s1):
    _, d = y.shape
    n = s0.shape[0]
    b_per_w = n // _NWORKERS
    nch = b_per_w // _SCW
    mesh = plsc.VectorSubcoreMesh(core_axis_name="c", subcore_axis_name="s")
    otype = jax.ShapeDtypeStruct((n, d), y.dtype)

    @functools.partial(
        pl.kernel, mesh=mesh,
        out_type=(otype, otype),
        scratch_types=[
            pltpu.VMEM((_SCW,), jnp.int32),
            pltpu.VMEM((_SCW,), jnp.int32),
            pltpu.VMEM((_SCW, d), jnp.float32),
            pltpu.VMEM((_SCW, d), jnp.float32),
            pltpu.SemaphoreType.DMA,
            pltpu.SemaphoreType.DMA,
            pltpu.SemaphoreType.DMA,
            pltpu.SemaphoreType.DMA,
        ])
    def k(y_hbm, s0_hbm, s1_hbm, o0_hbm, o1_hbm, ia, ib, ra, rb,
          sga, sgb, ssa, ssb):
        wid = jax.lax.axis_index("s") * 2 + jax.lax.axis_index("c")
        base = wid * b_per_w

        @pl.loop(0, nch)
        def _(j):
            off = base + j * _SCW
            pltpu.sync_copy(s0_hbm.at[pl.ds(off, _SCW)], ia)
            ga = pltpu.async_copy(y_hbm.at[ia], ra, sga)
            pltpu.sync_copy(s1_hbm.at[pl.ds(off, _SCW)], ib)
            gb = pltpu.async_copy(y_hbm.at[ib], rb, sgb)
            ga.wait()
            sa = pltpu.async_copy(ra, o0_hbm.at[pl.ds(off, _SCW)], ssa)
            gb.wait()
            sb = pltpu.async_copy(rb, o1_hbm.at[pl.ds(off, _SCW)], ssb)
            sa.wait()
            sb.wait()

    return k(y, s0, s1)


def _ffn_body(te_ref, nu_ref, x_ref, w1_ref, b1_ref, w2_ref, b2_ref, y_ref,
              w1b_ref, w2b_ref, last_ref):
    i = pl.program_id(0)

    @pl.when(i < nu_ref[0])
    def _():
        @pl.when((i == 0) | (te_ref[i] != last_ref[0]))
        def _():
            w1b_ref[...] = w1_ref[0].astype(jnp.bfloat16)
            w2b_ref[...] = w2_ref[0].astype(jnp.bfloat16)

        last_ref[0] = te_ref[i]
        xb = x_ref[...].astype(jnp.bfloat16)
        h = jax.lax.dot_general(xb, w1b_ref[...], (((1,), (0,)), ((), ())),
                                preferred_element_type=jnp.float32)
        h = h + b1_ref[0]
        h = 0.5 * h * (1.0 + jax.lax.erf(h * 0.7071067811865476))
        y = jax.lax.dot_general(h.astype(jnp.bfloat16), w2b_ref[...],
                                (((1,), (0,)), ((), ())),
                                preferred_element_type=jnp.float32)
        y_ref[...] = y + b2_ref[0]


def _run_ffn(x_sorted, tile_expert, n_used, W1, b1, W2, b2, nt_pad):
    e, d, hd = W1.shape
    grid_spec = pltpu.PrefetchScalarGridSpec(
        num_scalar_prefetch=2,
        grid=(nt_pad,),
        in_specs=[
            pl.BlockSpec((_TK, d), lambda i, te, nu: (i, 0)),
            pl.BlockSpec((1, d, hd), lambda i, te, nu: (te[i], 0, 0)),
            pl.BlockSpec((1, 1, hd), lambda i, te, nu: (te[i], 0, 0)),
            pl.BlockSpec((1, hd, d), lambda i, te, nu: (te[i], 0, 0)),
            pl.BlockSpec((1, 1, d), lambda i, te, nu: (te[i], 0, 0)),
        ],
        out_specs=pl.BlockSpec((_TK, d), lambda i, te, nu: (i, 0)),
        scratch_shapes=[
            pltpu.VMEM((d, hd), jnp.bfloat16),
            pltpu.VMEM((hd, d), jnp.bfloat16),
            pltpu.SMEM((1,), jnp.int32),
        ],
    )
    return pl.pallas_call(
        _ffn_body,
        grid_spec=grid_spec,
        out_shape=jax.ShapeDtypeStruct((nt_pad * _TK, d), jnp.float32),
        compiler_params=pltpu.CompilerParams(
            dimension_semantics=("arbitrary",)),
    )(tile_expert, n_used, x_sorted, W1, b1.reshape(e, 1, hd), W2,
      b2.reshape(e, 1, d))


def _combine_body(g_ref, y0_ref, y1_ref, o_ref):
    g = g_ref[...]
    o_ref[...] = g[:, 0:1] * y0_ref[...] + g[:, 1:2] * y1_ref[...]


def _run_combine(g, y0, y1):
    n, d = y0.shape
    nt = n // _RT
    return pl.pallas_call(
        _combine_body,
        grid=(nt,),
        in_specs=[
            pl.BlockSpec((_RT, 2), lambda i: (i, 0)),
            pl.BlockSpec((_RT, d), lambda i: (i, 0)),
            pl.BlockSpec((_RT, d), lambda i: (i, 0)),
        ],
        out_specs=pl.BlockSpec((_RT, d), lambda i: (i, 0)),
        out_shape=jax.ShapeDtypeStruct((n, d), jnp.float32),
    )(g, y0, y1)


@jax.jit
def kernel(x, Wg, bg, W1, b1, W2, b2):
    Bx, Tx, Dx = x.shape
    E, D, H = W1.shape
    xf = x.reshape(-1, Dx)
    N = xf.shape[0]
    nt_pad = (2 * N) // _TK + E
    p_slots = nt_pad * _TK

    g, e_idx, rank, counts = _run_router(xf, Wg, bg)

    cnt = counts[0]
    padded = ((cnt + _TK - 1) // _TK) * _TK
    ends = jnp.cumsum(padded)
    offs = ends - padded
    slots = jnp.take(offs, e_idx, axis=0) + rank
    s0 = slots[:, 0].astype(jnp.int32)
    s1 = slots[:, 1].astype(jnp.int32)
    n_used = (ends[E - 1] // _TK).astype(jnp.int32).reshape(1)
    tiles = jnp.arange(nt_pad, dtype=jnp.int32)
    te_raw = jnp.sum((ends[None, :] <= (tiles * _TK)[:, None]).astype(
        jnp.int32), axis=1)
    te_last = te_raw[jnp.clip(n_used[0] - 1, 0, nt_pad - 1)]
    tile_expert = jnp.clip(
        jnp.where(tiles >= n_used[0], te_last, te_raw),
        0, E - 1).astype(jnp.int32)

    x_sorted = _sc_dispatch(xf, s0, s1, p_slots)
    y = _run_ffn(x_sorted, tile_expert, n_used, W1, b1, W2, b2, nt_pad)
    y0, y1 = _sc_combine_gather(y, s0, s1)
    out = _run_combine(g, y0, y1)
    return out.reshape(Bx, Tx, Dx)
